# Initial kernel scaffold; baseline (speedup 1.0000x reference)
import jax
import jax.numpy as jnp
from jax.experimental import pallas as pl


def kernel(z, u, batch, batch_size, W, b):
    raise NotImplementedError("write your pallas kernel here")


# speedup vs baseline: 2.4248x; 2.4248x over previous
import functools

import jax
import jax.numpy as jnp
from jax import lax
from jax.experimental import pallas as pl
from jax.experimental.pallas import tpu as pltpu
from jax.experimental.pallas import tpu_sc as plsc

N, D = 50000, 256
B = 128
CHUNK = 80
NCHUNKS = N // CHUNK
NC, NS = 2, 16
NW = NC * NS
MAXK = (NCHUNKS + NW - 1) // NW
GROUPS = CHUNK // 16
CW = 16


def _sc_segment_sum_body(z_hbm, batch_hbm, sums_out, counts_out,
                         idx_v, rows_v, acc_v, cnt_v):
    cid = lax.axis_index("c")
    sid = lax.axis_index("s")
    wid = sid * NC + cid

    zeros16 = jnp.zeros((16,), jnp.float32)
    ones16 = jnp.ones((16,), jnp.float32)

    def zero_acc(i, c):
        acc_v[lax.div(i, D // 16), pl.ds(16 * lax.rem(i, D // 16), 16)] = zeros16
        return c
    lax.fori_loop(0, B * (D // 16), zero_acc, 0)

    def zero_cnt(i, c):
        cnt_v[i, :] = zeros16
        return c
    lax.fori_loop(0, B, zero_cnt, 0)

    start = (wid * NCHUNKS) // NW
    end = ((wid + 1) * NCHUNKS) // NW
    count = end - start

    def chunk_step(k, carry):
        @pl.when(k < count)
        def _():
            base = (start + k) * CHUNK
            pltpu.sync_copy(batch_hbm.at[pl.ds(base, CHUNK)], idx_v)
            pltpu.sync_copy(z_hbm.at[pl.ds(base, CHUNK)], rows_v)

            def group_step(g, c):
                iv = idx_v[pl.ds(g * 16, 16)]
                r0 = g * 16
                for j in range(16):
                    seg = iv[j]
                    for i in range(D // 16):
                        sl = pl.ds(16 * i, 16)
                        acc_v[seg, sl] = acc_v[seg, sl] + rows_v[r0 + j, sl]
                    cnt_v[seg, :] = cnt_v[seg, :] + ones16
                return c
            lax.fori_loop(0, GROUPS, group_step, 0)
        return carry

    lax.fori_loop(0, MAXK, chunk_step, 0)

    pltpu.sync_copy(acc_v, sums_out.at[wid])
    pltpu.sync_copy(cnt_v, counts_out.at[wid])


@functools.partial(
    pl.kernel,
    out_type=[
        jax.ShapeDtypeStruct((NW, B, D), jnp.float32),
        jax.ShapeDtypeStruct((NW, B, CW), jnp.float32),
    ],
    mesh=plsc.VectorSubcoreMesh(core_axis_name="c", subcore_axis_name="s"),
    scratch_types=[
        pltpu.VMEM((CHUNK,), jnp.int32),
        pltpu.VMEM((CHUNK, D), jnp.float32),
        pltpu.VMEM((B, D), jnp.float32),
        pltpu.VMEM((B, CW), jnp.float32),
    ],
)
def _sc_segment_sum(*refs):
    _sc_segment_sum_body(*refs)


def _tc_finish_body(psums_ref, pcnt_ref, u_ref, w_ref, b_ref, out_ref):
    sums = jnp.sum(psums_ref[...], axis=0)
    counts = jnp.sum(pcnt_ref[...], axis=0)[:, 0]
    graph = sums / jnp.maximum(counts, 1.0)[:, None]
    glob = jnp.dot(u_ref[...], w_ref[...], preferred_element_type=jnp.float32)
    glob = jnp.maximum(glob + b_ref[...], 0.0)
    out_ref[...] = jnp.concatenate([graph, glob], axis=-1)


def kernel(z, u, batch, batch_size, W, b):
    del batch_size
    psums, pcnt = _sc_segment_sum(z, batch.astype(jnp.int32))
    out = pl.pallas_call(
        _tc_finish_body,
        out_shape=jax.ShapeDtypeStruct((B, 2 * D), jnp.float32),
    )(psums, pcnt, u, W, b.reshape(1, D))
    return out


# speedup vs baseline: 4.4636x; 1.8408x over previous
import functools

import jax
import jax.numpy as jnp
from jax import lax
from jax.experimental import pallas as pl
from jax.experimental.pallas import tpu as pltpu
from jax.experimental.pallas import tpu_sc as plsc

N, D = 50000, 256
B = 128
CHUNK = 80
NCHUNKS = N // CHUNK
NC, NS = 2, 16
NW = NC * NS
MAXK = (NCHUNKS + NW - 1) // NW
GROUPS = CHUNK // 16
CW = 16


def _sc_segment_sum_body(z_hbm, batch_hbm, sums_out, counts_out,
                         idx_v, rows_v, acc_v, cnt_v):
    cid = lax.axis_index("c")
    sid = lax.axis_index("s")
    wid = sid * NC + cid

    zeros16 = jnp.zeros((16,), jnp.float32)
    ones16 = jnp.ones((16,), jnp.float32)

    def zero_acc(i, c):
        acc_v[lax.div(i, D // 16), pl.ds(16 * lax.rem(i, D // 16), 16)] = zeros16
        return c
    lax.fori_loop(0, B * (D // 16), zero_acc, 0)

    def zero_cnt(i, c):
        cnt_v[i, :] = zeros16
        return c
    lax.fori_loop(0, B, zero_cnt, 0)

    start = (wid * NCHUNKS) // NW
    end = ((wid + 1) * NCHUNKS) // NW
    count = end - start

    def chunk_step(k, carry):
        @pl.when(k < count)
        def _():
            base = (start + k) * CHUNK
            pltpu.sync_copy(batch_hbm.at[pl.ds(base, CHUNK)], idx_v)
            pltpu.sync_copy(z_hbm.at[pl.ds(base, CHUNK)], rows_v)

            def group_step(g, c):
                iv = idx_v[pl.ds(g * 16, 16)]
                r0 = g * 16
                seg0 = iv[0]
                uniform = seg0 == iv[15]

                @pl.when(uniform)
                def _():
                    for i in range(D // 16):
                        sl = pl.ds(16 * i, 16)
                        s01 = rows_v[r0 + 0, sl] + rows_v[r0 + 1, sl]
                        s23 = rows_v[r0 + 2, sl] + rows_v[r0 + 3, sl]
                        s45 = rows_v[r0 + 4, sl] + rows_v[r0 + 5, sl]
                        s67 = rows_v[r0 + 6, sl] + rows_v[r0 + 7, sl]
                        s89 = rows_v[r0 + 8, sl] + rows_v[r0 + 9, sl]
                        sab = rows_v[r0 + 10, sl] + rows_v[r0 + 11, sl]
                        scd = rows_v[r0 + 12, sl] + rows_v[r0 + 13, sl]
                        sef = rows_v[r0 + 14, sl] + rows_v[r0 + 15, sl]
                        s = ((s01 + s23) + (s45 + s67)) + (
                            (s89 + sab) + (scd + sef))
                        acc_v[seg0, sl] = acc_v[seg0, sl] + s
                    cnt_v[seg0, :] = cnt_v[seg0, :] + ones16 * 16.0

                @pl.when(jnp.logical_not(uniform))
                def _():
                    for j in range(16):
                        seg = iv[j]
                        for i in range(D // 16):
                            sl = pl.ds(16 * i, 16)
                            acc_v[seg, sl] = acc_v[seg, sl] + rows_v[r0 + j, sl]
                        cnt_v[seg, :] = cnt_v[seg, :] + ones16
                return c
            lax.fori_loop(0, GROUPS, group_step, 0)
        return carry

    lax.fori_loop(0, MAXK, chunk_step, 0)

    pltpu.sync_copy(acc_v, sums_out.at[wid])
    pltpu.sync_copy(cnt_v, counts_out.at[wid])


@functools.partial(
    pl.kernel,
    out_type=[
        jax.ShapeDtypeStruct((NW, B, D), jnp.float32),
        jax.ShapeDtypeStruct((NW, B, CW), jnp.float32),
    ],
    mesh=plsc.VectorSubcoreMesh(core_axis_name="c", subcore_axis_name="s"),
    scratch_types=[
        pltpu.VMEM((CHUNK,), jnp.int32),
        pltpu.VMEM((CHUNK, D), jnp.float32),
        pltpu.VMEM((B, D), jnp.float32),
        pltpu.VMEM((B, CW), jnp.float32),
    ],
)
def _sc_segment_sum(*refs):
    _sc_segment_sum_body(*refs)


def _tc_finish_body(psums_ref, pcnt_ref, u_ref, w_ref, b_ref, out_ref):
    sums = jnp.sum(psums_ref[...], axis=0)
    counts = jnp.sum(pcnt_ref[...], axis=0)[:, 0]
    graph = sums / jnp.maximum(counts, 1.0)[:, None]
    glob = jnp.dot(u_ref[...], w_ref[...], preferred_element_type=jnp.float32)
    glob = jnp.maximum(glob + b_ref[...], 0.0)
    out_ref[...] = jnp.concatenate([graph, glob], axis=-1)


def kernel(z, u, batch, batch_size, W, b):
    del batch_size
    psums, pcnt = _sc_segment_sum(z, batch.astype(jnp.int32))
    out = pl.pallas_call(
        _tc_finish_body,
        out_shape=jax.ShapeDtypeStruct((B, 2 * D), jnp.float32),
    )(psums, pcnt, u, W, b.reshape(1, D))
    return out


# speedup vs baseline: 6.5032x; 1.4569x over previous
import functools

import jax
import jax.numpy as jnp
from jax import lax
from jax.experimental import pallas as pl
from jax.experimental.pallas import tpu as pltpu
from jax.experimental.pallas import tpu_sc as plsc

N, D = 50000, 256
B = 128
CHUNK = 80
NCHUNKS = N // CHUNK
NC, NS = 2, 16
NW = NC * NS
MAXK = (NCHUNKS + NW - 1) // NW
GROUPS = CHUNK // 16
CW = 16


def _sc_segment_sum_body(z_hbm, batch_hbm, sums_out, counts_out,
                         idx_all, rows_v0, rows_v1, acc_v, cnt_v,
                         sem0, sem1):
    cid = lax.axis_index("c")
    sid = lax.axis_index("s")
    wid = sid * NC + cid

    zeros16 = jnp.zeros((16,), jnp.float32)
    ones16 = jnp.ones((16,), jnp.float32)

    start = (wid * NCHUNKS) // NW
    end = ((wid + 1) * NCHUNKS) // NW
    count = end - start

    rows = (rows_v0, rows_v1)
    sems = (sem0, sem1)

    def dma(c, buf):
        return pltpu.make_async_copy(
            z_hbm.at[pl.ds((start + c) * CHUNK, CHUNK)],
            rows[buf], sems[buf])

    dma(0, 0).start()
    pltpu.sync_copy(batch_hbm.at[pl.ds(start * CHUNK, MAXK * CHUNK)], idx_all)

    def zero_acc(i, c):
        for k in range(D // 16):
            acc_v[i, pl.ds(16 * k, 16)] = zeros16
        return c
    lax.fori_loop(0, B, zero_acc, 0)

    def zero_cnt(i, c):
        for k in range(8):
            cnt_v[8 * i + k, :] = zeros16
        return c
    lax.fori_loop(0, B // 8, zero_cnt, 0)

    def compute(c, rows_v):
        def group_step(g, cc):
            iv = idx_all[pl.ds(c * CHUNK + g * 16, 16)]
            r0 = g * 16
            seg0 = iv[0]
            uniform = seg0 == iv[15]

            @pl.when(uniform)
            def _():
                for i in range(D // 16):
                    sl = pl.ds(16 * i, 16)
                    s01 = rows_v[r0 + 0, sl] + rows_v[r0 + 1, sl]
                    s23 = rows_v[r0 + 2, sl] + rows_v[r0 + 3, sl]
                    s45 = rows_v[r0 + 4, sl] + rows_v[r0 + 5, sl]
                    s67 = rows_v[r0 + 6, sl] + rows_v[r0 + 7, sl]
                    s89 = rows_v[r0 + 8, sl] + rows_v[r0 + 9, sl]
                    sab = rows_v[r0 + 10, sl] + rows_v[r0 + 11, sl]
                    scd = rows_v[r0 + 12, sl] + rows_v[r0 + 13, sl]
                    sef = rows_v[r0 + 14, sl] + rows_v[r0 + 15, sl]
                    s = ((s01 + s23) + (s45 + s67)) + (
                        (s89 + sab) + (scd + sef))
                    acc_v[seg0, sl] = acc_v[seg0, sl] + s
                cnt_v[seg0, :] = cnt_v[seg0, :] + ones16 * 16.0

            @pl.when(jnp.logical_not(uniform))
            def _():
                for j in range(16):
                    seg = iv[j]
                    for i in range(D // 16):
                        sl = pl.ds(16 * i, 16)
                        acc_v[seg, sl] = acc_v[seg, sl] + rows_v[r0 + j, sl]
                    cnt_v[seg, :] = cnt_v[seg, :] + ones16
            return cc
        lax.fori_loop(0, GROUPS, group_step, 0)

    def pair_step(k, carry):
        for b in range(2):
            c = 2 * k + b
            nxt = c + 1

            @pl.when(nxt < count)
            def _():
                dma(nxt, 1 - b).start()

            @pl.when(c < count)
            def _():
                dma(c, b).wait()
                compute(c, rows[b])
        return carry

    lax.fori_loop(0, MAXK // 2, pair_step, 0)

    pltpu.sync_copy(acc_v, sums_out.at[wid])
    pltpu.sync_copy(cnt_v, counts_out.at[wid])


@functools.partial(
    pl.kernel,
    out_type=[
        jax.ShapeDtypeStruct((NW, B, D), jnp.float32),
        jax.ShapeDtypeStruct((NW, B, CW), jnp.float32),
    ],
    mesh=plsc.VectorSubcoreMesh(core_axis_name="c", subcore_axis_name="s"),
    scratch_types=[
        pltpu.VMEM((MAXK * CHUNK,), jnp.int32),
        pltpu.VMEM((CHUNK, D), jnp.float32),
        pltpu.VMEM((CHUNK, D), jnp.float32),
        pltpu.VMEM((B, D), jnp.float32),
        pltpu.VMEM((B, CW), jnp.float32),
        pltpu.SemaphoreType.DMA,
        pltpu.SemaphoreType.DMA,
    ],
)
def _sc_segment_sum(*refs):
    _sc_segment_sum_body(*refs)


def _tc_finish_body(psums_ref, pcnt_ref, u_ref, w_ref, b_ref, out_ref):
    sums = jnp.sum(psums_ref[...], axis=0)
    counts = jnp.sum(pcnt_ref[...], axis=0)[:, 0]
    graph = sums / jnp.maximum(counts, 1.0)[:, None]
    glob = jnp.dot(u_ref[...], w_ref[...], preferred_element_type=jnp.float32)
    glob = jnp.maximum(glob + b_ref[...], 0.0)
    out_ref[...] = jnp.concatenate([graph, glob], axis=-1)


def kernel(z, u, batch, batch_size, W, b):
    del batch_size
    psums, pcnt = _sc_segment_sum(z, batch.astype(jnp.int32))
    out = pl.pallas_call(
        _tc_finish_body,
        out_shape=jax.ShapeDtypeStruct((B, 2 * D), jnp.float32),
    )(psums, pcnt, u, W, b.reshape(1, D))
    return out
